# Initial kernel scaffold; baseline (speedup 1.0000x reference)
import jax
import jax.numpy as jnp
from jax.experimental import pallas as pl


def kernel(xyz, W0, b0, g0, be0, W1, b1, g1, be1, W2, b2, g2, be2):
    raise NotImplementedError("write your pallas kernel here")


# speedup vs baseline: 13.3649x; 13.3649x over previous
import functools

import jax
import jax.numpy as jnp
from jax import lax
from jax.experimental import pallas as pl
from jax.experimental.pallas import tpu as pltpu
from jax.experimental.pallas import tpu_sc as plsc

B = 8
N = 4096
Q = 1024
K = 32
P = B * Q * K
EPSV = 1e-5
BIG = 1e30


def _fps_body(xyzT_ref, far0_ref, new_ref):
    xp = xyzT_ref[0]
    yp = xyzT_ref[1]
    zp = xyzT_ref[2]
    col = lax.broadcasted_iota(jnp.int32, (B, N), 1)

    def body(i, carry):
        dmin, far = carry
        mask = col == far
        cx = jnp.sum(jnp.where(mask, xp, 0.0), axis=1, keepdims=True)
        cy = jnp.sum(jnp.where(mask, yp, 0.0), axis=1, keepdims=True)
        cz = jnp.sum(jnp.where(mask, zp, 0.0), axis=1, keepdims=True)
        new_ref[:, pl.ds(i, 1), 0:1] = cx[:, :, None]
        new_ref[:, pl.ds(i, 1), 1:2] = cy[:, :, None]
        new_ref[:, pl.ds(i, 1), 2:3] = cz[:, :, None]
        dist = (xp - cx) ** 2 + (yp - cy) ** 2 + (zp - cz) ** 2
        dmin = jnp.where(dist < dmin, dist, dmin)
        m = jnp.max(dmin, axis=1, keepdims=True)
        far = jnp.min(jnp.where(dmin == m, col, N), axis=1, keepdims=True)
        return dmin, far

    dmin0 = jnp.full((B, N), 1e10, dtype=jnp.float32)
    lax.fori_loop(0, Q, body, (dmin0, far0_ref[...]))


def _fps(xyzT, far0):
    return pl.pallas_call(
        _fps_body,
        out_shape=jax.ShapeDtypeStruct((B, Q, 3), jnp.float32),
    )(xyzT, far0)


def _knn_body(nc_ref, nr_ref, idx_ref, d_ref):
    rx = nr_ref[0].reshape(1, Q)
    ry = nr_ref[1].reshape(1, Q)
    rz = nr_ref[2].reshape(1, Q)
    RB = 128
    NB = Q // RB
    for rb in range(NB):
        cx = nc_ref[0, pl.ds(rb * RB, RB), 0:1]
        cy = nc_ref[0, pl.ds(rb * RB, RB), 1:2]
        cz = nc_ref[0, pl.ds(rb * RB, RB), 2:3]
        riota = lax.broadcasted_iota(jnp.int32, (RB, Q), 0) + rb * RB
        ciota = lax.broadcasted_iota(jnp.int32, (RB, Q), 1)
        d = (cx - rx) ** 2 + (cy - ry) ** 2 + (cz - rz) ** 2
        d_ref[pl.ds(rb * RB, RB), :] = jnp.where(riota == ciota, BIG, d)

    def round_body(kk, _):
        def min_pass(rb, acc):
            db = d_ref[pl.ds(rb * RB, RB), :]
            return jnp.minimum(acc, jnp.min(db, axis=0, keepdims=True))

        m = lax.fori_loop(0, NB, min_pass,
                          jnp.full((1, Q), BIG, dtype=jnp.float32))

        def arg_pass(rb, acc):
            db = d_ref[pl.ds(rb * RB, RB), :]
            ri = lax.broadcasted_iota(jnp.int32, (RB, Q), 0) + rb * RB
            cand = jnp.min(jnp.where(db == m, ri, N), axis=0, keepdims=True)
            return jnp.minimum(acc, cand)

        im = lax.fori_loop(0, NB, arg_pass,
                           jnp.full((1, Q), N, dtype=jnp.int32))
        idx_ref[0, pl.ds(kk, 1), :] = im

        def mask_pass(rb, c):
            db = d_ref[pl.ds(rb * RB, RB), :]
            ri = lax.broadcasted_iota(jnp.int32, (RB, Q), 0) + rb * RB
            d_ref[pl.ds(rb * RB, RB), :] = jnp.where(ri == im, BIG, db)
            return c

        return lax.fori_loop(0, NB, mask_pass, _)

    lax.fori_loop(0, K, round_body, 0)


def _knn(new_xyz, newT):
    return pl.pallas_call(
        _knn_body,
        grid=(B,),
        in_specs=[
            pl.BlockSpec((1, Q, 3), lambda b: (b, 0, 0)),
            pl.BlockSpec((3, 1, 1, Q), lambda b: (0, b, 0, 0)),
        ],
        out_specs=pl.BlockSpec((1, K, Q), lambda b: (b, 0, 0)),
        out_shape=jax.ShapeDtypeStruct((B, K, Q), jnp.int32),
        scratch_shapes=[pltpu.VMEM((Q, Q), jnp.float32)],
    )(new_xyz, newT.reshape(3, B, 1, Q))


QC = 256
NCHUNK = Q // QC


def _sc_gather_body(xyzT_hbm, newT_hbm, idxT_hbm, out_hbm,
                    xv, yv, zv, nxv, nyv, nzv, idx_v, ox, oy, oz):
    wid = lax.axis_index("s") * 2 + lax.axis_index("c")
    b = wid // NCHUNK
    qlo = (wid % NCHUNK) * QC

    planes = (xv, yv, zv)
    nplanes = (nxv, nyv, nzv)
    oplanes = (ox, oy, oz)
    for c in range(3):
        pltpu.sync_copy(xyzT_hbm.at[c, b, :], planes[c])
        pltpu.sync_copy(newT_hbm.at[c, b, pl.ds(qlo, QC)], nplanes[c])
    pltpu.sync_copy(idxT_hbm.at[b, :, pl.ds(qlo, QC)], idx_v)

    def body(t, carry):
        k = t // (QC // 16)
        qv = t % (QC // 16)
        iv = idx_v[k, pl.ds(qv * 16, 16)]
        for c in range(3):
            g = plsc.load_gather(planes[c], [iv])
            nv = nplanes[c][pl.ds(qv * 16, 16)]
            oplanes[c][k, pl.ds(qv * 16, 16)] = g - nv
        return carry

    lax.fori_loop(0, K * (QC // 16), body, 0)

    for c in range(3):
        pltpu.sync_copy(oplanes[c], out_hbm.at[c, b, :, pl.ds(qlo, QC)])


def _sc_gather(xyzT, newT, idxT):
    mesh = plsc.VectorSubcoreMesh(core_axis_name="c", subcore_axis_name="s")
    f = pl.kernel(
        _sc_gather_body,
        out_type=jax.ShapeDtypeStruct((3, B, K, Q), jnp.float32),
        mesh=mesh,
        compiler_params=pltpu.CompilerParams(needs_layout_passes=False),
        scratch_types=[
            pltpu.VMEM((N,), jnp.float32),
            pltpu.VMEM((N,), jnp.float32),
            pltpu.VMEM((N,), jnp.float32),
            pltpu.VMEM((QC,), jnp.float32),
            pltpu.VMEM((QC,), jnp.float32),
            pltpu.VMEM((QC,), jnp.float32),
            pltpu.VMEM((K, QC), jnp.int32),
            pltpu.VMEM((K, QC), jnp.float32),
            pltpu.VMEM((K, QC), jnp.float32),
            pltpu.VMEM((K, QC), jnp.float32),
        ],
    )
    return f(xyzT, newT, idxT)


TL = 2048
G = P // TL


def _stats0_body(x_ref, s1_ref, s2_ref, s1a, s2a):
    pid = pl.program_id(0)

    @pl.when(pid == 0)
    def _():
        s1a[...] = jnp.zeros_like(s1a)
        s2a[...] = jnp.zeros_like(s2a)

    x = x_ref[...]
    s1a[...] += jnp.sum(x, axis=1, keepdims=True)
    s2a[...] += lax.dot_general(x, x, (((1,), (1,)), ((), ())),
                                preferred_element_type=jnp.float32)

    @pl.when(pid == G - 1)
    def _():
        s1_ref[...] = s1a[...]
        s2_ref[...] = s2a[...]


def _stats0(x):
    cin = x.shape[0]
    return pl.pallas_call(
        _stats0_body,
        grid=(G,),
        in_specs=[pl.BlockSpec((cin, TL), lambda i: (0, i))],
        out_specs=[
            pl.BlockSpec((cin, 1), lambda i: (0, 0)),
            pl.BlockSpec((cin, cin), lambda i: (0, 0)),
        ],
        out_shape=[
            jax.ShapeDtypeStruct((cin, 1), jnp.float32),
            jax.ShapeDtypeStruct((cin, cin), jnp.float32),
        ],
        scratch_shapes=[pltpu.VMEM((cin, 1), jnp.float32),
                        pltpu.VMEM((cin, cin), jnp.float32)],
    )(x)


def _layer_body(x_ref, w_ref, b_ref, z_ref, s1_ref, s2_ref, s1a, s2a):
    pid = pl.program_id(0)

    @pl.when(pid == 0)
    def _():
        s1a[...] = jnp.zeros_like(s1a)
        s2a[...] = jnp.zeros_like(s2a)

    x = x_ref[...]
    w = w_ref[...]
    z = jnp.maximum(
        lax.dot_general(w, x, (((1,), (0,)), ((), ())),
                        preferred_element_type=jnp.float32) + b_ref[...], 0.0)
    z_ref[...] = z
    s1a[...] += jnp.sum(z, axis=1, keepdims=True)
    s2a[...] += lax.dot_general(z, z, (((1,), (1,)), ((), ())),
                                preferred_element_type=jnp.float32)

    @pl.when(pid == G - 1)
    def _():
        s1_ref[...] = s1a[...]
        s2_ref[...] = s2a[...]


def _layer(x, wf, bf):
    cin = x.shape[0]
    cout = wf.shape[0]
    return pl.pallas_call(
        _layer_body,
        grid=(G,),
        in_specs=[
            pl.BlockSpec((cin, TL), lambda i: (0, i)),
            pl.BlockSpec((cout, cin), lambda i: (0, 0)),
            pl.BlockSpec((cout, 1), lambda i: (0, 0)),
        ],
        out_specs=[
            pl.BlockSpec((cout, TL), lambda i: (0, i)),
            pl.BlockSpec((cout, 1), lambda i: (0, 0)),
            pl.BlockSpec((cout, cout), lambda i: (0, 0)),
        ],
        out_shape=[
            jax.ShapeDtypeStruct((cout, P), jnp.float32),
            jax.ShapeDtypeStruct((cout, 1), jnp.float32),
            jax.ShapeDtypeStruct((cout, cout), jnp.float32),
        ],
        scratch_shapes=[pltpu.VMEM((cout, 1), jnp.float32),
                        pltpu.VMEM((cout, cout), jnp.float32)],
    )(x, wf, bf)


def _l2max_body(z_ref, w_ref, b_ref, o_ref):
    w = w_ref[...]
    bb = b_ref[...]
    m = jnp.full((w.shape[0], z_ref.shape[3]), -BIG, dtype=jnp.float32)
    for k in range(K):
        zk = z_ref[:, 0, k, :]
        y = jnp.maximum(
            lax.dot_general(w, zk, (((1,), (0,)), ((), ())),
                            preferred_element_type=jnp.float32) + bb, 0.0)
        m = jnp.maximum(m, y)
    o_ref[0] = m


def _l2max(z4, wf, bf):
    cin = z4.shape[0]
    cout = wf.shape[0]
    QT = 256
    return pl.pallas_call(
        _l2max_body,
        grid=(B, Q // QT),
        in_specs=[
            pl.BlockSpec((cin, 1, K, QT), lambda b, q: (0, b, 0, q)),
            pl.BlockSpec((cout, cin), lambda b, q: (0, 0)),
            pl.BlockSpec((cout, 1), lambda b, q: (0, 0)),
        ],
        out_specs=pl.BlockSpec((1, cout, QT), lambda b, q: (b, 0, q)),
        out_shape=jax.ShapeDtypeStruct((B, cout, Q), jnp.float32),
    )(z4, wf, bf)


def _fold(w, bias, gamma, beta, s1, s2):
    m = s1[:, 0] / P
    sig = s2 / P - jnp.outer(m, m)
    mean_y = w @ m + bias
    var_y = jnp.einsum('oi,ij,oj->o', w, sig, w)
    inv = gamma / jnp.sqrt(var_y + EPSV)
    return w * inv[:, None], (inv * (bias - mean_y) + beta)[:, None]


def kernel(xyz, W0, b0, g0, be0, W1, b1, g1, be1, W2, b2, g2, be2):
    far0 = jax.random.randint(jax.random.key(1), (B,), 0, N)
    far0 = far0.astype(jnp.int32)[:, None]
    xyzT = jnp.transpose(xyz, (2, 0, 1))
    new_xyz = _fps(xyzT, far0)
    newT = jnp.transpose(new_xyz, (2, 0, 1))
    idxT = _knn(new_xyz, newT)
    x0 = _sc_gather(xyzT, newT, idxT)
    x0f = x0.reshape(3, P)
    s1, s2 = _stats0(x0f)
    wf, bf = _fold(W0, b0, g0, be0, s1, s2)
    z1, s1, s2 = _layer(x0f, wf, bf)
    wf, bf = _fold(W1, b1, g1, be1, s1, s2)
    z2, s1, s2 = _layer(z1, wf, bf)
    wf, bf = _fold(W2, b2, g2, be2, s1, s2)
    new_points = _l2max(z2.reshape(64, B, K, Q), wf, bf)
    return new_xyz, new_points


# speedup vs baseline: 14.5377x; 1.0878x over previous
import functools

import jax
import jax.numpy as jnp
from jax import lax
from jax.experimental import pallas as pl
from jax.experimental.pallas import tpu as pltpu
from jax.experimental.pallas import tpu_sc as plsc

B = 8
N = 4096
Q = 1024
K = 32
P = B * Q * K
EPSV = 1e-5
BIG = 1e30


def _fps_body(xyzT_ref, far0_ref, new_ref):
    xp = xyzT_ref[0]
    yp = xyzT_ref[1]
    zp = xyzT_ref[2]
    col = lax.broadcasted_iota(jnp.int32, (B, N), 1)

    def body(i, carry):
        dmin, far = carry
        mask = col == far
        cx = jnp.sum(jnp.where(mask, xp, 0.0), axis=1, keepdims=True)
        cy = jnp.sum(jnp.where(mask, yp, 0.0), axis=1, keepdims=True)
        cz = jnp.sum(jnp.where(mask, zp, 0.0), axis=1, keepdims=True)
        new_ref[:, pl.ds(i, 1), :] = jnp.concatenate(
            [cx, cy, cz], axis=1)[:, None, :]
        dist = (xp - cx) ** 2 + (yp - cy) ** 2 + (zp - cz) ** 2
        dmin = jnp.where(dist < dmin, dist, dmin)
        m = jnp.max(dmin, axis=1, keepdims=True)
        far = jnp.min(jnp.where(dmin == m, col, N), axis=1, keepdims=True)
        return dmin, far

    dmin0 = jnp.full((B, N), 1e10, dtype=jnp.float32)
    lax.fori_loop(0, Q, body, (dmin0, far0_ref[...]))


def _fps(xyzT, far0):
    return pl.pallas_call(
        _fps_body,
        out_shape=jax.ShapeDtypeStruct((B, Q, 3), jnp.float32),
    )(xyzT, far0)


def _knn_body(nc_ref, nr_ref, idx_ref, d_ref):
    rx = nr_ref[0].reshape(1, Q)
    ry = nr_ref[1].reshape(1, Q)
    rz = nr_ref[2].reshape(1, Q)
    RB = 128
    NB = Q // RB
    for rb in range(NB):
        cx = nc_ref[0, pl.ds(rb * RB, RB), 0:1]
        cy = nc_ref[0, pl.ds(rb * RB, RB), 1:2]
        cz = nc_ref[0, pl.ds(rb * RB, RB), 2:3]
        riota = lax.broadcasted_iota(jnp.int32, (RB, Q), 0) + rb * RB
        ciota = lax.broadcasted_iota(jnp.int32, (RB, Q), 1)
        d = (cx - rx) ** 2 + (cy - ry) ** 2 + (cz - rz) ** 2
        d_ref[pl.ds(rb * RB, RB), :] = jnp.where(riota == ciota, BIG, d)

    def round_body(kk, _):
        def min_pass(rb, acc):
            db = d_ref[pl.ds(rb * RB, RB), :]
            return jnp.minimum(acc, jnp.min(db, axis=0, keepdims=True))

        m = lax.fori_loop(0, NB, min_pass,
                          jnp.full((1, Q), BIG, dtype=jnp.float32))

        def arg_mask_pass(rb, acc):
            db = d_ref[pl.ds(rb * RB, RB), :]
            ri = lax.broadcasted_iota(jnp.int32, (RB, Q), 0) + rb * RB
            hit = db == m
            d_ref[pl.ds(rb * RB, RB), :] = jnp.where(hit, BIG, db)
            cand = jnp.min(jnp.where(hit, ri, N), axis=0, keepdims=True)
            return jnp.minimum(acc, cand)

        im = lax.fori_loop(0, NB, arg_mask_pass,
                           jnp.full((1, Q), N, dtype=jnp.int32))
        idx_ref[0, pl.ds(kk, 1), :] = im
        return _

    lax.fori_loop(0, K, round_body, 0)


def _knn(new_xyz, newT):
    return pl.pallas_call(
        _knn_body,
        grid=(B,),
        in_specs=[
            pl.BlockSpec((1, Q, 3), lambda b: (b, 0, 0)),
            pl.BlockSpec((3, 1, 1, Q), lambda b: (0, b, 0, 0)),
        ],
        out_specs=pl.BlockSpec((1, K, Q), lambda b: (b, 0, 0)),
        out_shape=jax.ShapeDtypeStruct((B, K, Q), jnp.int32),
        scratch_shapes=[pltpu.VMEM((Q, Q), jnp.float32)],
    )(new_xyz, newT.reshape(3, B, 1, Q))


QC = 256
NCHUNK = Q // QC


def _sc_gather_body(xyzT_hbm, newT_hbm, idxT_hbm, out_hbm,
                    xv, yv, zv, nxv, nyv, nzv, idx_v, ox, oy, oz):
    wid = lax.axis_index("s") * 2 + lax.axis_index("c")
    b = wid // NCHUNK
    qlo = (wid % NCHUNK) * QC

    planes = (xv, yv, zv)
    nplanes = (nxv, nyv, nzv)
    oplanes = (ox, oy, oz)
    for c in range(3):
        pltpu.sync_copy(xyzT_hbm.at[c, b, :], planes[c])
        pltpu.sync_copy(newT_hbm.at[c, b, pl.ds(qlo, QC)], nplanes[c])
    pltpu.sync_copy(idxT_hbm.at[b, :, pl.ds(qlo, QC)], idx_v)

    def body(t, carry):
        k = t // (QC // 16)
        qv = t % (QC // 16)
        iv = idx_v[k, pl.ds(qv * 16, 16)]
        for c in range(3):
            g = plsc.load_gather(planes[c], [iv])
            nv = nplanes[c][pl.ds(qv * 16, 16)]
            oplanes[c][k, pl.ds(qv * 16, 16)] = g - nv
        return carry

    lax.fori_loop(0, K * (QC // 16), body, 0)

    for c in range(3):
        pltpu.sync_copy(oplanes[c], out_hbm.at[c, b, :, pl.ds(qlo, QC)])


def _sc_gather(xyzT, newT, idxT):
    mesh = plsc.VectorSubcoreMesh(core_axis_name="c", subcore_axis_name="s")
    f = pl.kernel(
        _sc_gather_body,
        out_type=jax.ShapeDtypeStruct((3, B, K, Q), jnp.float32),
        mesh=mesh,
        compiler_params=pltpu.CompilerParams(needs_layout_passes=False),
        scratch_types=[
            pltpu.VMEM((N,), jnp.float32),
            pltpu.VMEM((N,), jnp.float32),
            pltpu.VMEM((N,), jnp.float32),
            pltpu.VMEM((QC,), jnp.float32),
            pltpu.VMEM((QC,), jnp.float32),
            pltpu.VMEM((QC,), jnp.float32),
            pltpu.VMEM((K, QC), jnp.int32),
            pltpu.VMEM((K, QC), jnp.float32),
            pltpu.VMEM((K, QC), jnp.float32),
            pltpu.VMEM((K, QC), jnp.float32),
        ],
    )
    return f(xyzT, newT, idxT)


TL = 2048
G = P // TL


def _stats0_body(x_ref, s1_ref, s2_ref, s1a, s2a):
    pid = pl.program_id(0)

    @pl.when(pid == 0)
    def _():
        s1a[...] = jnp.zeros_like(s1a)
        s2a[...] = jnp.zeros_like(s2a)

    x = x_ref[...]
    s1a[...] += jnp.sum(x, axis=1, keepdims=True)
    s2a[...] += lax.dot_general(x, x, (((1,), (1,)), ((), ())),
                                preferred_element_type=jnp.float32)

    @pl.when(pid == G - 1)
    def _():
        s1_ref[...] = s1a[...]
        s2_ref[...] = s2a[...]


def _stats0(x):
    cin = x.shape[0]
    return pl.pallas_call(
        _stats0_body,
        grid=(G,),
        in_specs=[pl.BlockSpec((cin, TL), lambda i: (0, i))],
        out_specs=[
            pl.BlockSpec((cin, 1), lambda i: (0, 0)),
            pl.BlockSpec((cin, cin), lambda i: (0, 0)),
        ],
        out_shape=[
            jax.ShapeDtypeStruct((cin, 1), jnp.float32),
            jax.ShapeDtypeStruct((cin, cin), jnp.float32),
        ],
        scratch_shapes=[pltpu.VMEM((cin, 1), jnp.float32),
                        pltpu.VMEM((cin, cin), jnp.float32)],
    )(x)


def _layer_body(x_ref, w_ref, b_ref, z_ref, s1_ref, s2_ref, s1a, s2a):
    pid = pl.program_id(0)

    @pl.when(pid == 0)
    def _():
        s1a[...] = jnp.zeros_like(s1a)
        s2a[...] = jnp.zeros_like(s2a)

    x = x_ref[...]
    w = w_ref[...]
    z = jnp.maximum(
        lax.dot_general(w, x, (((1,), (0,)), ((), ())),
                        preferred_element_type=jnp.float32) + b_ref[...], 0.0)
    z_ref[...] = z
    s1a[...] += jnp.sum(z, axis=1, keepdims=True)
    s2a[...] += lax.dot_general(z, z, (((1,), (1,)), ((), ())),
                                preferred_element_type=jnp.float32)

    @pl.when(pid == G - 1)
    def _():
        s1_ref[...] = s1a[...]
        s2_ref[...] = s2a[...]


def _layer(x, wf, bf):
    cin = x.shape[0]
    cout = wf.shape[0]
    return pl.pallas_call(
        _layer_body,
        grid=(G,),
        in_specs=[
            pl.BlockSpec((cin, TL), lambda i: (0, i)),
            pl.BlockSpec((cout, cin), lambda i: (0, 0)),
            pl.BlockSpec((cout, 1), lambda i: (0, 0)),
        ],
        out_specs=[
            pl.BlockSpec((cout, TL), lambda i: (0, i)),
            pl.BlockSpec((cout, 1), lambda i: (0, 0)),
            pl.BlockSpec((cout, cout), lambda i: (0, 0)),
        ],
        out_shape=[
            jax.ShapeDtypeStruct((cout, P), jnp.float32),
            jax.ShapeDtypeStruct((cout, 1), jnp.float32),
            jax.ShapeDtypeStruct((cout, cout), jnp.float32),
        ],
        scratch_shapes=[pltpu.VMEM((cout, 1), jnp.float32),
                        pltpu.VMEM((cout, cout), jnp.float32)],
    )(x, wf, bf)


def _l2max_body(z_ref, w_ref, b_ref, o_ref):
    w = w_ref[...]
    bb = b_ref[...]
    m = jnp.full((w.shape[0], z_ref.shape[3]), -BIG, dtype=jnp.float32)
    for k in range(K):
        zk = z_ref[:, 0, k, :]
        y = jnp.maximum(
            lax.dot_general(w, zk, (((1,), (0,)), ((), ())),
                            preferred_element_type=jnp.float32) + bb, 0.0)
        m = jnp.maximum(m, y)
    o_ref[0] = m


def _l2max(z4, wf, bf):
    cin = z4.shape[0]
    cout = wf.shape[0]
    QT = 256
    return pl.pallas_call(
        _l2max_body,
        grid=(B, Q // QT),
        in_specs=[
            pl.BlockSpec((cin, 1, K, QT), lambda b, q: (0, b, 0, q)),
            pl.BlockSpec((cout, cin), lambda b, q: (0, 0)),
            pl.BlockSpec((cout, 1), lambda b, q: (0, 0)),
        ],
        out_specs=pl.BlockSpec((1, cout, QT), lambda b, q: (b, 0, q)),
        out_shape=jax.ShapeDtypeStruct((B, cout, Q), jnp.float32),
    )(z4, wf, bf)


def _fold(w, bias, gamma, beta, s1, s2):
    m = s1[:, 0] / P
    sig = s2 / P - jnp.outer(m, m)
    mean_y = w @ m + bias
    var_y = jnp.einsum('oi,ij,oj->o', w, sig, w)
    inv = gamma / jnp.sqrt(var_y + EPSV)
    return w * inv[:, None], (inv * (bias - mean_y) + beta)[:, None]


def kernel(xyz, W0, b0, g0, be0, W1, b1, g1, be1, W2, b2, g2, be2):
    far0 = jax.random.randint(jax.random.key(1), (B,), 0, N)
    far0 = far0.astype(jnp.int32)[:, None]
    xyzT = jnp.transpose(xyz, (2, 0, 1))
    new_xyz = _fps(xyzT, far0)
    newT = jnp.transpose(new_xyz, (2, 0, 1))
    idxT = _knn(new_xyz, newT)
    x0 = _sc_gather(xyzT, newT, idxT)
    x0f = x0.reshape(3, P)
    s1, s2 = _stats0(x0f)
    wf, bf = _fold(W0, b0, g0, be0, s1, s2)
    z1, s1, s2 = _layer(x0f, wf, bf)
    wf, bf = _fold(W1, b1, g1, be1, s1, s2)
    z2, s1, s2 = _layer(z1, wf, bf)
    wf, bf = _fold(W2, b2, g2, be2, s1, s2)
    new_points = _l2max(z2.reshape(64, B, K, Q), wf, bf)
    return new_xyz, new_points


# speedup vs baseline: 15.9003x; 1.0937x over previous
import functools

import jax
import jax.numpy as jnp
from jax import lax
from jax.experimental import pallas as pl
from jax.experimental.pallas import tpu as pltpu
from jax.experimental.pallas import tpu_sc as plsc

B = 8
N = 4096
Q = 1024
K = 32
P = B * Q * K
EPSV = 1e-5
BIG = 1e30


def _fps_body(xyz24_ref, far0_ref, new_ref):
    col = lax.broadcasted_iota(jnp.int32, (B, N), 1)
    col24 = lax.broadcasted_iota(jnp.int32, (3 * B, N), 1)

    def body(i, carry):
        dmin, far = carry
        a24 = xyz24_ref[...]
        far24 = jnp.concatenate([far, far, far], axis=0)
        s24 = jnp.sum(jnp.where(col24 == far24, a24, 0.0),
                      axis=1, keepdims=True)
        new_ref[:, pl.ds(i, 1), :] = jnp.concatenate(
            [s24[0:B], s24[B:2 * B], s24[2 * B:]], axis=1)[:, None, :]
        dist = ((a24[0:B] - s24[0:B]) ** 2
                + (a24[B:2 * B] - s24[B:2 * B]) ** 2
                + (a24[2 * B:] - s24[2 * B:]) ** 2)
        dmin = jnp.minimum(dmin, dist)
        m = jnp.max(dmin, axis=1, keepdims=True)
        far = jnp.min(jnp.where(dmin == m, col, N), axis=1, keepdims=True)
        return dmin, far

    dmin0 = jnp.full((B, N), 1e10, dtype=jnp.float32)
    lax.fori_loop(0, Q, body, (dmin0, far0_ref[...]))


def _fps(xyz24, far0):
    return pl.pallas_call(
        _fps_body,
        out_shape=jax.ShapeDtypeStruct((B, Q, 3), jnp.float32),
    )(xyz24, far0)


def _knn_body(nc_ref, nr_ref, idx_ref, d_ref):
    rx = nr_ref[0].reshape(1, Q)
    ry = nr_ref[1].reshape(1, Q)
    rz = nr_ref[2].reshape(1, Q)
    RB = 128
    NB = Q // RB
    for rb in range(NB):
        cx = nc_ref[0, pl.ds(rb * RB, RB), 0:1]
        cy = nc_ref[0, pl.ds(rb * RB, RB), 1:2]
        cz = nc_ref[0, pl.ds(rb * RB, RB), 2:3]
        riota = lax.broadcasted_iota(jnp.int32, (RB, Q), 0) + rb * RB
        ciota = lax.broadcasted_iota(jnp.int32, (RB, Q), 1)
        d = (cx - rx) ** 2 + (cy - ry) ** 2 + (cz - rz) ** 2
        d_ref[pl.ds(rb * RB, RB), :] = jnp.where(riota == ciota, BIG, d)

    def round_body(kk, _):
        def min_pass(rb, acc):
            db = d_ref[pl.ds(rb * RB, RB), :]
            return jnp.minimum(acc, jnp.min(db, axis=0, keepdims=True))

        m = lax.fori_loop(0, NB, min_pass,
                          jnp.full((1, Q), BIG, dtype=jnp.float32))

        def arg_mask_pass(rb, acc):
            db = d_ref[pl.ds(rb * RB, RB), :]
            ri = lax.broadcasted_iota(jnp.int32, (RB, Q), 0) + rb * RB
            hit = db == m
            d_ref[pl.ds(rb * RB, RB), :] = jnp.where(hit, BIG, db)
            cand = jnp.min(jnp.where(hit, ri, N), axis=0, keepdims=True)
            return jnp.minimum(acc, cand)

        im = lax.fori_loop(0, NB, arg_mask_pass,
                           jnp.full((1, Q), N, dtype=jnp.int32))
        idx_ref[0, pl.ds(kk, 1), :] = im
        return _

    lax.fori_loop(0, K, round_body, 0)


def _knn(new_xyz, newT):
    return pl.pallas_call(
        _knn_body,
        grid=(B,),
        in_specs=[
            pl.BlockSpec((1, Q, 3), lambda b: (b, 0, 0)),
            pl.BlockSpec((3, 1, 1, Q), lambda b: (0, b, 0, 0)),
        ],
        out_specs=pl.BlockSpec((1, K, Q), lambda b: (b, 0, 0)),
        out_shape=jax.ShapeDtypeStruct((B, K, Q), jnp.int32),
        scratch_shapes=[pltpu.VMEM((Q, Q), jnp.float32)],
    )(new_xyz, newT.reshape(3, B, 1, Q))


QC = 256
NCHUNK = Q // QC


def _sc_gather_body(xyzT_hbm, newT_hbm, idxT_hbm, out_hbm,
                    xv, yv, zv, nxv, nyv, nzv, idx_v, ox, oy, oz):
    wid = lax.axis_index("s") * 2 + lax.axis_index("c")
    b = wid // NCHUNK
    qlo = (wid % NCHUNK) * QC

    planes = (xv, yv, zv)
    nplanes = (nxv, nyv, nzv)
    oplanes = (ox, oy, oz)
    for c in range(3):
        pltpu.sync_copy(xyzT_hbm.at[c, b, :], planes[c])
        pltpu.sync_copy(newT_hbm.at[c, b, pl.ds(qlo, QC)], nplanes[c])
    pltpu.sync_copy(idxT_hbm.at[b, :, pl.ds(qlo, QC)], idx_v)

    def body(t, carry):
        k = t // (QC // 16)
        qv = t % (QC // 16)
        iv = idx_v[k, pl.ds(qv * 16, 16)]
        for c in range(3):
            g = plsc.load_gather(planes[c], [iv])
            nv = nplanes[c][pl.ds(qv * 16, 16)]
            oplanes[c][k, pl.ds(qv * 16, 16)] = g - nv
        return carry

    lax.fori_loop(0, K * (QC // 16), body, 0)

    for c in range(3):
        pltpu.sync_copy(oplanes[c], out_hbm.at[c, b, :, pl.ds(qlo, QC)])


def _sc_gather(xyzT, newT, idxT):
    mesh = plsc.VectorSubcoreMesh(core_axis_name="c", subcore_axis_name="s")
    f = pl.kernel(
        _sc_gather_body,
        out_type=jax.ShapeDtypeStruct((3, B, K, Q), jnp.float32),
        mesh=mesh,
        compiler_params=pltpu.CompilerParams(needs_layout_passes=False),
        scratch_types=[
            pltpu.VMEM((N,), jnp.float32),
            pltpu.VMEM((N,), jnp.float32),
            pltpu.VMEM((N,), jnp.float32),
            pltpu.VMEM((QC,), jnp.float32),
            pltpu.VMEM((QC,), jnp.float32),
            pltpu.VMEM((QC,), jnp.float32),
            pltpu.VMEM((K, QC), jnp.int32),
            pltpu.VMEM((K, QC), jnp.float32),
            pltpu.VMEM((K, QC), jnp.float32),
            pltpu.VMEM((K, QC), jnp.float32),
        ],
    )
    return f(xyzT, newT, idxT)


TL = 4096
G = P // TL


def _stats0_body(x_ref, s1_ref, s2_ref, s1a, s2a):
    pid = pl.program_id(0)

    @pl.when(pid == 0)
    def _():
        s1a[...] = jnp.zeros_like(s1a)
        s2a[...] = jnp.zeros_like(s2a)

    x = x_ref[...]
    s1a[...] += jnp.sum(x, axis=1, keepdims=True)
    s2a[...] += lax.dot_general(x, x, (((1,), (1,)), ((), ())),
                                preferred_element_type=jnp.float32)

    @pl.when(pid == G - 1)
    def _():
        s1_ref[...] = s1a[...]
        s2_ref[...] = s2a[...]


def _stats0(x):
    cin = x.shape[0]
    return pl.pallas_call(
        _stats0_body,
        grid=(G,),
        in_specs=[pl.BlockSpec((cin, TL), lambda i: (0, i))],
        out_specs=[
            pl.BlockSpec((cin, 1), lambda i: (0, 0)),
            pl.BlockSpec((cin, cin), lambda i: (0, 0)),
        ],
        out_shape=[
            jax.ShapeDtypeStruct((cin, 1), jnp.float32),
            jax.ShapeDtypeStruct((cin, cin), jnp.float32),
        ],
        scratch_shapes=[pltpu.VMEM((cin, 1), jnp.float32),
                        pltpu.VMEM((cin, cin), jnp.float32)],
    )(x)


def _layer_body(x_ref, w_ref, b_ref, z_ref, s1_ref, s2_ref, s1a, s2a):
    pid = pl.program_id(0)

    @pl.when(pid == 0)
    def _():
        s1a[...] = jnp.zeros_like(s1a)
        s2a[...] = jnp.zeros_like(s2a)

    x = x_ref[...]
    w = w_ref[...]
    z = jnp.maximum(
        lax.dot_general(w, x, (((1,), (0,)), ((), ())),
                        preferred_element_type=jnp.float32) + b_ref[...], 0.0)
    z_ref[...] = z
    s1a[...] += jnp.sum(z, axis=1, keepdims=True)
    s2a[...] += lax.dot_general(z, z, (((1,), (1,)), ((), ())),
                                preferred_element_type=jnp.float32)

    @pl.when(pid == G - 1)
    def _():
        s1_ref[...] = s1a[...]
        s2_ref[...] = s2a[...]


def _layer(x, wf, bf):
    cin = x.shape[0]
    cout = wf.shape[0]
    return pl.pallas_call(
        _layer_body,
        grid=(G,),
        in_specs=[
            pl.BlockSpec((cin, TL), lambda i: (0, i)),
            pl.BlockSpec((cout, cin), lambda i: (0, 0)),
            pl.BlockSpec((cout, 1), lambda i: (0, 0)),
        ],
        out_specs=[
            pl.BlockSpec((cout, TL), lambda i: (0, i)),
            pl.BlockSpec((cout, 1), lambda i: (0, 0)),
            pl.BlockSpec((cout, cout), lambda i: (0, 0)),
        ],
        out_shape=[
            jax.ShapeDtypeStruct((cout, P), jnp.float32),
            jax.ShapeDtypeStruct((cout, 1), jnp.float32),
            jax.ShapeDtypeStruct((cout, cout), jnp.float32),
        ],
        scratch_shapes=[pltpu.VMEM((cout, 1), jnp.float32),
                        pltpu.VMEM((cout, cout), jnp.float32)],
    )(x, wf, bf)


def _l2max_body(z_ref, w_ref, b_ref, o_ref):
    w = w_ref[...]
    bb = b_ref[...]
    m = jnp.full((w.shape[0], z_ref.shape[3]), -BIG, dtype=jnp.float32)
    for k in range(K):
        zk = z_ref[:, 0, k, :]
        y = jnp.maximum(
            lax.dot_general(w, zk, (((1,), (0,)), ((), ())),
                            preferred_element_type=jnp.float32) + bb, 0.0)
        m = jnp.maximum(m, y)
    o_ref[0] = m


def _l2max(z4, wf, bf):
    cin = z4.shape[0]
    cout = wf.shape[0]
    QT = 256
    return pl.pallas_call(
        _l2max_body,
        grid=(B, Q // QT),
        in_specs=[
            pl.BlockSpec((cin, 1, K, QT), lambda b, q: (0, b, 0, q)),
            pl.BlockSpec((cout, cin), lambda b, q: (0, 0)),
            pl.BlockSpec((cout, 1), lambda b, q: (0, 0)),
        ],
        out_specs=pl.BlockSpec((1, cout, QT), lambda b, q: (b, 0, q)),
        out_shape=jax.ShapeDtypeStruct((B, cout, Q), jnp.float32),
    )(z4, wf, bf)


def _fold(w, bias, gamma, beta, s1, s2):
    m = s1[:, 0] / P
    sig = s2 / P - jnp.outer(m, m)
    mean_y = w @ m + bias
    var_y = jnp.einsum('oi,ij,oj->o', w, sig, w)
    inv = gamma / jnp.sqrt(var_y + EPSV)
    return w * inv[:, None], (inv * (bias - mean_y) + beta)[:, None]


def kernel(xyz, W0, b0, g0, be0, W1, b1, g1, be1, W2, b2, g2, be2):
    far0 = jax.random.randint(jax.random.key(1), (B,), 0, N)
    far0 = far0.astype(jnp.int32)[:, None]
    xyzT = jnp.transpose(xyz, (2, 0, 1))
    new_xyz = _fps(xyzT.reshape(3 * B, N), far0)
    newT = jnp.transpose(new_xyz, (2, 0, 1))
    idxT = _knn(new_xyz, newT)
    x0 = _sc_gather(xyzT, newT, idxT)
    x0f = x0.reshape(3, P)
    s1, s2 = _stats0(x0f)
    wf, bf = _fold(W0, b0, g0, be0, s1, s2)
    z1, s1, s2 = _layer(x0f, wf, bf)
    wf, bf = _fold(W1, b1, g1, be1, s1, s2)
    z2, s1, s2 = _layer(z1, wf, bf)
    wf, bf = _fold(W2, b2, g2, be2, s1, s2)
    new_points = _l2max(z2.reshape(64, B, K, Q), wf, bf)
    return new_xyz, new_points
